# baseline (speedup 1.0000x reference)
import jax
import jax.numpy as jnp
from jax import lax
from jax.experimental import pallas as pl
from jax.experimental.pallas import tpu as pltpu
from jax.experimental.pallas import tpu_sc as plsc

VOCAB_ROWS = 1000000
BATCH = 16384
FIELDS = 26
N_EMBED = 64
NUM_IDX = BATCH * FIELDS

NUM_CORES = 2
NUM_SUBCORES = 16
NUM_WORKERS = NUM_CORES * NUM_SUBCORES

K_BATCH = 8
W_ROWS = K_BATCH * FIELDS
WINDOWS_PER_WORKER = BATCH // (K_BATCH * NUM_WORKERS)

PAD_BLOCK = 8000


def _pad_to_128(emb):
    def copy_body(in_ref, out_ref):
        out_ref[:, 0:N_EMBED] = in_ref[...]

    return pl.pallas_call(
        copy_body,
        grid=(VOCAB_ROWS // PAD_BLOCK,),
        in_specs=[pl.BlockSpec((PAD_BLOCK, N_EMBED), lambda i: (i, 0))],
        out_specs=pl.BlockSpec((PAD_BLOCK, 2 * N_EMBED), lambda i: (i, 0)),
        out_shape=jax.ShapeDtypeStruct((VOCAB_ROWS, 2 * N_EMBED), jnp.float32),
    )(emb)


def _sc_gather(scr, flat_idx):
    mesh = plsc.VectorSubcoreMesh(core_axis_name="core", subcore_axis_name="subcore")

    @pl.kernel(
        out_type=jax.ShapeDtypeStruct((BATCH, FIELDS, N_EMBED), jnp.float32),
        mesh=mesh,
        scratch_types=[
            pltpu.VMEM((W_ROWS,), jnp.int32),
            pltpu.VMEM((W_ROWS, 2 * N_EMBED), jnp.float32),
            pltpu.VMEM((W_ROWS, 2 * N_EMBED), jnp.float32),
            pltpu.VMEM((W_ROWS, N_EMBED), jnp.float32),
            pltpu.VMEM((W_ROWS, N_EMBED), jnp.float32),
            pltpu.SemaphoreType.DMA,
            pltpu.SemaphoreType.DMA,
        ],
    )
    def kern(scr_hbm, idx_hbm, out_hbm, idx_v, g_v0, g_v1, o_v0, o_v1, sem0, sem1):
        wid = lax.axis_index("core") * NUM_SUBCORES + lax.axis_index("subcore")

        def do_window(w, g_v, o_v, sem, drain):
            b0 = (wid * WINDOWS_PER_WORKER + w) * K_BATCH
            @pl.when(drain)
            def _():
                for j in range(K_BATCH):
                    pltpu.make_async_copy(
                        o_v.at[pl.ds(j * FIELDS, FIELDS)],
                        out_hbm.at[0],
                        sem,
                    ).wait()

            pltpu.sync_copy(idx_hbm.at[wid * WINDOWS_PER_WORKER + w], idx_v)
            pltpu.sync_copy(scr_hbm.at[idx_v], g_v)
            o_v[...] = g_v[:, 0:N_EMBED]
            for j in range(K_BATCH):
                pltpu.async_copy(
                    o_v.at[pl.ds(j * FIELDS, FIELDS)],
                    out_hbm.at[b0 + j],
                    sem,
                )

        @pl.loop(0, WINDOWS_PER_WORKER, step=2)
        def _(w):
            do_window(w, g_v0, o_v0, sem0, w >= 2)
            do_window(w + 1, g_v1, o_v1, sem1, w >= 2)

        for o_v, sem in ((o_v0, sem0), (o_v1, sem1)):
            for j in range(K_BATCH):
                pltpu.make_async_copy(
                    o_v.at[pl.ds(j * FIELDS, FIELDS)],
                    out_hbm.at[0],
                    sem,
                ).wait()

    return kern(scr, flat_idx)


def kernel(discrete, embeddings):
    flat_idx = discrete.astype(jnp.int32).reshape(NUM_IDX // W_ROWS, W_ROWS)
    scr = _pad_to_128(embeddings)
    return _sc_gather(scr, flat_idx)


# speedup vs baseline: 1.0749x; 1.0749x over previous
import jax
import jax.numpy as jnp
from jax import lax
from jax.experimental import pallas as pl
from jax.experimental.pallas import tpu as pltpu
from jax.experimental.pallas import tpu_sc as plsc

VOCAB_ROWS = 1000000
BATCH = 16384
FIELDS = 26
N_EMBED = 64
NUM_IDX = BATCH * FIELDS

NUM_CORES = 2
NUM_SUBCORES = 16
NUM_WORKERS = NUM_CORES * NUM_SUBCORES

K_BATCH = 8
W_ROWS = K_BATCH * FIELDS
WINDOWS_PER_WORKER = BATCH // (K_BATCH * NUM_WORKERS)

PAD_BLOCK = 20000


def _pad_to_128(emb):
    def copy_body(in_ref, out_ref):
        out_ref[:, 0:N_EMBED] = in_ref[...]

    return pl.pallas_call(
        copy_body,
        grid=(VOCAB_ROWS // PAD_BLOCK,),
        in_specs=[pl.BlockSpec((PAD_BLOCK, N_EMBED), lambda i: (i, 0))],
        out_specs=pl.BlockSpec((PAD_BLOCK, 2 * N_EMBED), lambda i: (i, 0)),
        out_shape=jax.ShapeDtypeStruct((VOCAB_ROWS, 2 * N_EMBED), jnp.float32),
    )(emb)


def _sc_gather(scr, idx_windows):
    mesh = plsc.VectorSubcoreMesh(core_axis_name="core", subcore_axis_name="subcore")

    @pl.kernel(
        out_type=jax.ShapeDtypeStruct((BATCH, FIELDS, N_EMBED), jnp.float32),
        mesh=mesh,
        scratch_types=[
            pltpu.VMEM((W_ROWS,), jnp.int32),
            pltpu.VMEM((W_ROWS,), jnp.int32),
            pltpu.VMEM((W_ROWS, 2 * N_EMBED), jnp.float32),
            pltpu.VMEM((W_ROWS, 2 * N_EMBED), jnp.float32),
            pltpu.VMEM((W_ROWS, N_EMBED), jnp.float32),
            pltpu.VMEM((W_ROWS, N_EMBED), jnp.float32),
            pltpu.SemaphoreType.DMA,
            pltpu.SemaphoreType.DMA,
            pltpu.SemaphoreType.DMA,
            pltpu.SemaphoreType.DMA,
        ],
    )
    def kern(scr_hbm, idx_hbm, out_hbm,
             idx_v0, idx_v1, g_v0, g_v1, o_v0, o_v1,
             gsem0, gsem1, osem0, osem1):
        wid = lax.axis_index("core") * NUM_SUBCORES + lax.axis_index("subcore")
        w_base = wid * WINDOWS_PER_WORKER

        bufs = ((idx_v0, g_v0, o_v0, gsem0, osem0),
                (idx_v1, g_v1, o_v1, gsem1, osem1))

        def start_gather(w, idx_v, g_v, gsem):
            pltpu.sync_copy(idx_hbm.at[w_base + w], idx_v)
            pltpu.async_copy(scr_hbm.at[idx_v], g_v, gsem)

        for b in range(2):
            idx_v, g_v, _, gsem, _ = bufs[b]
            start_gather(b, idx_v, g_v, gsem)

        def do_window(w, idx_v, g_v, o_v, gsem, osem):
            b0 = (w_base + w) * K_BATCH
            pltpu.make_async_copy(scr_hbm.at[idx_v], g_v, gsem).wait()
            @pl.when(w >= 2)
            def _():
                for j in range(K_BATCH):
                    pltpu.make_async_copy(
                        o_v.at[pl.ds(j * FIELDS, FIELDS)], out_hbm.at[0], osem
                    ).wait()

            o_v[...] = g_v[:, 0:N_EMBED]
            @pl.when(w + 2 < WINDOWS_PER_WORKER)
            def _():
                start_gather(w + 2, idx_v, g_v, gsem)

            for j in range(K_BATCH):
                pltpu.async_copy(
                    o_v.at[pl.ds(j * FIELDS, FIELDS)], out_hbm.at[b0 + j], osem
                )

        @pl.loop(0, WINDOWS_PER_WORKER, step=2)
        def _(w):
            for b in range(2):
                idx_v, g_v, o_v, gsem, osem = bufs[b]
                do_window(w + b, idx_v, g_v, o_v, gsem, osem)

        for b in range(2):
            _, _, o_v, _, osem = bufs[b]
            for j in range(K_BATCH):
                pltpu.make_async_copy(
                    o_v.at[pl.ds(j * FIELDS, FIELDS)], out_hbm.at[0], osem
                ).wait()

    return kern(scr, idx_windows)


def kernel(discrete, embeddings):
    idx_windows = discrete.astype(jnp.int32).reshape(NUM_IDX // W_ROWS, W_ROWS)
    scr = _pad_to_128(embeddings)
    return _sc_gather(scr, idx_windows)


# speedup vs baseline: 1.2131x; 1.1286x over previous
import jax
import jax.numpy as jnp
from jax import lax
from jax.experimental import pallas as pl
from jax.experimental.pallas import tpu as pltpu
from jax.experimental.pallas import tpu_sc as plsc

VOCAB_ROWS = 1000000
BATCH = 16384
FIELDS = 26
N_EMBED = 64
NUM_IDX = BATCH * FIELDS

NUM_CORES = 2
NUM_SUBCORES = 16
NUM_WORKERS = NUM_CORES * NUM_SUBCORES

K_BATCH = 8
W_ROWS = K_BATCH * FIELDS
WINDOWS_PER_WORKER = BATCH // (K_BATCH * NUM_WORKERS)

PAD_BLOCK = 20000


def _pad_to_128(emb):
    def copy_body(in_ref, out_ref):
        out_ref[:, 0:N_EMBED] = in_ref[...]

    return pl.pallas_call(
        copy_body,
        grid=(VOCAB_ROWS // PAD_BLOCK,),
        in_specs=[pl.BlockSpec((PAD_BLOCK, N_EMBED), lambda i: (i, 0))],
        out_specs=pl.BlockSpec((PAD_BLOCK, 2 * N_EMBED), lambda i: (i, 0)),
        out_shape=jax.ShapeDtypeStruct((VOCAB_ROWS, 2 * N_EMBED), jnp.float32),
    )(emb)


def _sc_gather(scr, idx_windows):
    mesh = plsc.VectorSubcoreMesh(core_axis_name="core", subcore_axis_name="subcore")

    @pl.kernel(
        out_type=jax.ShapeDtypeStruct((BATCH, FIELDS, N_EMBED), jnp.float32),
        mesh=mesh,
        scratch_types=[
            pltpu.VMEM((W_ROWS,), jnp.int32),
            pltpu.VMEM((W_ROWS,), jnp.int32),
            pltpu.VMEM((W_ROWS, 2 * N_EMBED), jnp.float32),
            pltpu.VMEM((W_ROWS, 2 * N_EMBED), jnp.float32),
            pltpu.VMEM((W_ROWS, N_EMBED), jnp.float32),
            pltpu.VMEM((W_ROWS, N_EMBED), jnp.float32),
            pltpu.SemaphoreType.DMA,
            pltpu.SemaphoreType.DMA,
            pltpu.SemaphoreType.DMA,
            pltpu.SemaphoreType.DMA,
        ],
    )
    def kern(scr_hbm, idx_hbm, out_hbm,
             idx_v0, idx_v1, g_v0, g_v1, o_v0, o_v1,
             gsem0, gsem1, osem0, osem1):
        wid = lax.axis_index("core") * NUM_SUBCORES + lax.axis_index("subcore")
        w_base = wid * WINDOWS_PER_WORKER

        bufs = ((idx_v0, g_v0, o_v0, gsem0, osem0),
                (idx_v1, g_v1, o_v1, gsem1, osem1))

        def start_gather(w, idx_v, g_v, gsem):
            pltpu.sync_copy(idx_hbm.at[w_base + w], idx_v)
            pltpu.async_copy(scr_hbm.at[idx_v], g_v, gsem)

        for b in range(2):
            idx_v, g_v, _, gsem, _ = bufs[b]
            start_gather(b, idx_v, g_v, gsem)

        def do_window(w, idx_v, g_v, o_v, gsem, osem):
            b0 = (w_base + w) * K_BATCH
            pltpu.make_async_copy(scr_hbm.at[idx_v], g_v, gsem).wait()
            @pl.when(w >= 2)
            def _():
                for j in range(K_BATCH):
                    pltpu.make_async_copy(
                        o_v.at[pl.ds(j * FIELDS, FIELDS)], out_hbm.at[0], osem
                    ).wait()

            o_v[...] = g_v[:, 0:N_EMBED]
            @pl.when(w + 2 < WINDOWS_PER_WORKER)
            def _():
                start_gather(w + 2, idx_v, g_v, gsem)

            for j in range(K_BATCH):
                pltpu.async_copy(
                    o_v.at[pl.ds(j * FIELDS, FIELDS)], out_hbm.at[b0 + j], osem
                )

        @pl.loop(0, WINDOWS_PER_WORKER, step=2)
        def _(w):
            for b in range(2):
                idx_v, g_v, o_v, gsem, osem = bufs[b]
                do_window(w + b, idx_v, g_v, o_v, gsem, osem)

        for b in range(2):
            _, _, o_v, _, osem = bufs[b]
            for j in range(K_BATCH):
                pltpu.make_async_copy(
                    o_v.at[pl.ds(j * FIELDS, FIELDS)], out_hbm.at[0], osem
                ).wait()

    return kern(scr, idx_windows)


def kernel(discrete, embeddings):
    idx_windows = discrete.astype(jnp.int32).reshape(NUM_IDX // W_ROWS, W_ROWS)
    scr = jnp.pad(embeddings, ((0, 0), (0, N_EMBED)))
    return _sc_gather(scr, idx_windows)
